# initial kernel scaffold (unmeasured)
import jax
import jax.numpy as jnp
from jax import lax
from jax.experimental import pallas as pl
from jax.experimental.pallas import tpu as pltpu


def kernel(
    x,
):
    def body(*refs):
        pass

    out_shape = jax.ShapeDtypeStruct(..., jnp.float32)
    return pl.pallas_call(body, out_shape=out_shape)(...)


# baseline (device time: 590083 ns/iter reference)
import jax
import jax.numpy as jnp
from jax import lax
from jax.experimental import pallas as pl
from jax.experimental.pallas import tpu as pltpu

jax.config.update("jax_compilation_cache_dir", "/tmp/jax_cache")
jax.config.update("jax_persistent_cache_min_compile_time_secs", 0.0)

N_DEV = 32
M = 2048
LOG_M = 11
LOG_TOTAL = 16
N_CROSS = 15


def kernel(x):
    m, n = x.shape
    assert m == M

    def body(x_ref, out_ref, wbuf, rbuf, send_sem, recv_sem, ready_sems):
        my_pos = lax.axis_index("i")

        wbuf[...] = x_ref[...].astype(jnp.bfloat16)

        i2d = lax.broadcasted_iota(jnp.int32, (M, 1), 0)
        g = my_pos.astype(jnp.int32) * M + i2d

        cross_idx = 0
        for L in range(1, LOG_TOTAL + 1):
            for dlog in range(L - 1, -1, -1):
                d = 1 << dlog
                asc = ((g >> L) & 1) == 0
                bit = (g & d) != 0
                keep_min = jnp.logical_xor(bit, asc)
                if d < M:
                    z = wbuf[...]
                    up = pltpu.roll(z, M - d, axis=0)
                    dn = pltpu.roll(z, d, axis=0)
                    partner = jnp.where(bit, dn, up)
                    mn = jnp.minimum(z, partner)
                    mx = jnp.maximum(z, partner)
                    wbuf[...] = jnp.where(keep_min, mn, mx)
                else:
                    delta = d >> LOG_M
                    partner_dev = my_pos ^ delta
                    pl.semaphore_signal(
                        ready_sems.at[cross_idx],
                        inc=1,
                        device_id=(partner_dev,),
                        device_id_type=pl.DeviceIdType.MESH,
                    )
                    pl.semaphore_wait(ready_sems.at[cross_idx], 1)
                    rdma = pltpu.make_async_remote_copy(
                        src_ref=wbuf,
                        dst_ref=rbuf,
                        send_sem=send_sem,
                        recv_sem=recv_sem,
                        device_id=(partner_dev,),
                        device_id_type=pl.DeviceIdType.MESH,
                    )
                    rdma.start()
                    rdma.wait()
                    z = wbuf[...]
                    r = rbuf[...]
                    mn = jnp.minimum(z, r)
                    mx = jnp.maximum(z, r)
                    wbuf[...] = jnp.where(keep_min, mn, mx)
                    cross_idx += 1

        out_ref[...] = wbuf[...].astype(jnp.float32)

    return pl.pallas_call(
        body,
        out_shape=jax.ShapeDtypeStruct((m, n), jnp.float32),
        in_specs=[pl.BlockSpec(memory_space=pltpu.VMEM)],
        out_specs=pl.BlockSpec(memory_space=pltpu.VMEM),
        scratch_shapes=[
            pltpu.VMEM((M, n), jnp.bfloat16),
            pltpu.VMEM((M, n), jnp.bfloat16),
            pltpu.SemaphoreType.DMA,
            pltpu.SemaphoreType.DMA,
            pltpu.SemaphoreType.REGULAR((N_CROSS,)),
        ],
    )(x)


# device time: 585287 ns/iter; 1.0082x vs baseline; 1.0082x over previous
import jax
import jax.numpy as jnp
from jax import lax
from jax.experimental import pallas as pl
from jax.experimental.pallas import tpu as pltpu

jax.config.update("jax_compilation_cache_dir", "/tmp/jax_cache")
jax.config.update("jax_persistent_cache_min_compile_time_secs", 0.0)

N_DEV = 32
M = 2048
LOG_M = 11
LOG_TOTAL = 16
N_CROSS = 15


def kernel(x):
    m, n = x.shape
    assert m == M

    cross_deltas = [
        1 << (dlog - LOG_M)
        for L in range(LOG_M + 1, LOG_TOTAL + 1)
        for dlog in range(L - 1, LOG_M - 1, -1)
    ]
    assert len(cross_deltas) == N_CROSS

    def body(x_ref, out_ref, wbuf, rbuf, send_sem, recv_sem, ready_sems):
        my_pos = lax.axis_index("i")

        wbuf[...] = x_ref[...].astype(jnp.bfloat16)

        i2d = lax.broadcasted_iota(jnp.int32, (M, 1), 0)
        g = my_pos.astype(jnp.int32) * M + i2d

        def grant_credit(k):
            pl.semaphore_signal(
                ready_sems.at[k],
                inc=1,
                device_id=(my_pos ^ cross_deltas[k],),
                device_id_type=pl.DeviceIdType.MESH,
            )

        grant_credit(0)

        cross_idx = 0
        for L in range(1, LOG_TOTAL + 1):
            for dlog in range(L - 1, -1, -1):
                d = 1 << dlog
                asc = ((g >> L) & 1) == 0
                bit = (g & d) != 0
                keep_min = jnp.logical_xor(bit, asc)
                if d < M:
                    z = wbuf[...]
                    up = pltpu.roll(z, M - d, axis=0)
                    dn = pltpu.roll(z, d, axis=0)
                    partner = jnp.where(bit, dn, up)
                    mn = jnp.minimum(z, partner)
                    mx = jnp.maximum(z, partner)
                    wbuf[...] = jnp.where(keep_min, mn, mx)
                else:
                    delta = d >> LOG_M
                    partner_dev = my_pos ^ delta
                    pl.semaphore_wait(ready_sems.at[cross_idx], 1)
                    rdma = pltpu.make_async_remote_copy(
                        src_ref=wbuf,
                        dst_ref=rbuf,
                        send_sem=send_sem,
                        recv_sem=recv_sem,
                        device_id=(partner_dev,),
                        device_id_type=pl.DeviceIdType.MESH,
                    )
                    rdma.start()
                    rdma.wait()
                    z = wbuf[...]
                    r = rbuf[...]
                    mn = jnp.minimum(z, r)
                    mx = jnp.maximum(z, r)
                    wbuf[...] = jnp.where(keep_min, mn, mx)
                    cross_idx += 1
                    if cross_idx < N_CROSS:
                        grant_credit(cross_idx)

        out_ref[...] = wbuf[...].astype(jnp.float32)

    return pl.pallas_call(
        body,
        out_shape=jax.ShapeDtypeStruct((m, n), jnp.float32),
        in_specs=[pl.BlockSpec(memory_space=pltpu.VMEM)],
        out_specs=pl.BlockSpec(memory_space=pltpu.VMEM),
        scratch_shapes=[
            pltpu.VMEM((M, n), jnp.bfloat16),
            pltpu.VMEM((M, n), jnp.bfloat16),
            pltpu.SemaphoreType.DMA,
            pltpu.SemaphoreType.DMA,
            pltpu.SemaphoreType.REGULAR((N_CROSS,)),
        ],
    )(x)
